# baseline (speedup 1.0000x reference)
import jax
import jax.numpy as jnp
from jax import lax
from jax.experimental import pallas as pl
from jax.experimental.pallas import tpu as pltpu
from jax.experimental.pallas import tpu_sc as plsc

N = 10000
E = 320000
DIN = 128
D1 = 64
D2 = 16

NC = 2
NS = 16
NW = NC * NS
CHUNK = 128
CPW = 80
EPW = CPW * CHUNK
EPAD = NW * EPW
NPAD = 10240
RPT = NPAD // NS
DEGW = 8

_mesh = plsc.VectorSubcoreMesh(
    core_axis_name="c", subcore_axis_name="s", num_cores=NC, num_subcores=NS
)
_sc_params = pltpu.CompilerParams(use_tc_tiling_on_sc=False,
                                  needs_layout_passes=False,
                                  disable_bounds_checks=True)


_DEG_GRP = 8


def _deg_body(idx, ones_hbm, zrows, out, acc, dstv, ones_v, sem):
  cid = lax.axis_index("c")
  sid = lax.axis_index("s")
  w = cid * NS + sid
  pltpu.sync_copy(idx.at[1, w], dstv)
  pltpu.sync_copy(ones_hbm, ones_v)
  pltpu.sync_copy(zrows, acc.at[pl.ds(sid * RPT, RPT)])
  plsc.subcore_barrier()

  @pl.loop(0, CPW, step=_DEG_GRP)
  def _(j):
    for b in range(_DEG_GRP):
      pltpu.async_copy(ones_v, acc.at[dstv.at[j + b]], sem, add=True)
    for b in range(_DEG_GRP):
      pltpu.make_async_copy(ones_v, acc.at[dstv.at[j + b]], sem).wait()

  plsc.subcore_barrier()
  pltpu.sync_copy(acc.at[pl.ds(sid * RPT, RPT)],
                  out.at[cid, pl.ds(sid * RPT, RPT)])


_deg_kernel = pl.kernel(
    _deg_body,
    out_type=jax.ShapeDtypeStruct((NC, NPAD, DEGW), jnp.float32),
    mesh=_mesh,
    compiler_params=_sc_params,
    scratch_types=[
        pltpu.VMEM_SHARED((NPAD, DEGW), jnp.float32),
        pltpu.VMEM((CPW, CHUNK), jnp.int32),
        pltpu.VMEM((CHUNK, DEGW), jnp.float32),
        pltpu.SemaphoreType.DMA,
    ],
)


def _make_segsum(D, NBUF, NPH):
  CW = CPW // NPH

  def body(idx, table, zrows, out, acc, tab, srcv, dstv, *bufsem):
    bufs = bufsem[:NBUF]
    gsems = bufsem[NBUF:2 * NBUF]
    ssems = bufsem[2 * NBUF:]
    cid = lax.axis_index("c")
    sid = lax.axis_index("s")
    w = cid * NS + sid
    rows = pl.ds(sid * RPT, RPT)
    pltpu.sync_copy(zrows, acc.at[rows])
    pltpu.sync_copy(table.at[rows], tab.at[rows])
    plsc.subcore_barrier()

    for ph in range(NPH):
      pltpu.sync_copy(idx.at[0, w, pl.ds(ph * CW, CW)], srcv)
      pltpu.sync_copy(idx.at[1, w, pl.ds(ph * CW, CW)], dstv)
      for b in range(NBUF):
        pltpu.async_copy(tab.at[srcv.at[b]], bufs[b], gsems[b])

      @pl.loop(0, CW, step=NBUF)
      def _(j):
        for b in range(NBUF):
          pltpu.make_async_copy(tab.at[srcv.at[j + b]], bufs[b],
                                gsems[b]).wait()
          pltpu.async_copy(bufs[b], acc.at[dstv.at[j + b]], ssems[b], add=True)
        for b in range(NBUF):
          pltpu.make_async_copy(bufs[b], acc.at[dstv.at[j + b]],
                                ssems[b]).wait()
          nj = j + b + NBUF

          @pl.when(nj < CW)
          def _():
            pltpu.async_copy(tab.at[srcv.at[nj]], bufs[b], gsems[b])

    plsc.subcore_barrier()
    pltpu.sync_copy(acc.at[rows], out.at[cid, rows])

  return pl.kernel(
      body,
      out_type=jax.ShapeDtypeStruct((NC, NPAD, D), jnp.float32),
      mesh=_mesh,
      compiler_params=_sc_params,
      scratch_types=[
          pltpu.VMEM_SHARED((NPAD, D), jnp.float32),
          pltpu.VMEM_SHARED((NPAD, D), jnp.float32),
          pltpu.VMEM((CPW // NPH, CHUNK), jnp.int32),
          pltpu.VMEM((CPW // NPH, CHUNK), jnp.int32),
      ] + [pltpu.VMEM((CHUNK, D), jnp.float32) for _ in range(NBUF)]
        + [pltpu.SemaphoreType.DMA for _ in range(2 * NBUF)],
  )


_segsum64 = _make_segsum(D1, 2, 1)
_segsum16 = _make_segsum(D2, 4, 1)


def _dec_body(idx, acc2, h2s, disw, b2w, a_out, z2_out,
              ztab, srcv, dstv, a0v, a1v, h2v, dv, b2v, obuf, *bufsem):
  bufs = bufsem[:8]
  sems = bufsem[8:]
  cid = lax.axis_index("c")
  sid = lax.axis_index("s")
  w = cid * NS + sid
  rows = pl.ds(sid * RPT, RPT)
  pltpu.sync_copy(idx.at[0, w], srcv)
  pltpu.sync_copy(idx.at[1, w], dstv)
  pltpu.sync_copy(acc2.at[0, rows], a0v)
  pltpu.sync_copy(acc2.at[1, rows], a1v)
  pltpu.sync_copy(h2s.at[rows], h2v)
  pltpu.sync_copy(disw.at[rows], dv)
  pltpu.sync_copy(b2w, b2v)
  b2vec = b2v[0, :]

  @pl.loop(0, RPT)
  def _(r):
    z = dv[r, :] * (a0v[r, :] + a1v[r, :] + h2v[r, :]) + b2vec
    a0v[r, :] = jnp.maximum(z, 0.0)

  pltpu.sync_copy(a0v, ztab.at[rows])
  pltpu.sync_copy(a0v, z2_out.at[rows])
  plsc.subcore_barrier()

  rings = tuple((bufs[2 * p], bufs[2 * p + 1], sems[2 * p], sems[2 * p + 1])
                for p in range(4))
  for p in range(4):
    bs, bd, ss, sd = rings[p]
    pltpu.async_copy(ztab.at[srcv.at[p]], bs, ss)
    pltpu.async_copy(ztab.at[dstv.at[p]], bd, sd)

  @pl.loop(0, CPW, step=4)
  def _(j):
    for p in range(4):
      bs, bd, ss, sd = rings[p]
      jj = j + p
      pltpu.make_async_copy(ztab.at[srcv.at[jj]], bs, ss).wait()
      pltpu.make_async_copy(ztab.at[dstv.at[jj]], bd, sd).wait()
      for g in range(CHUNK // 16):
        r16 = lax.iota(jnp.int32, 16) + g * 16
        parts = [jnp.zeros((16,), jnp.float32) for _ in range(4)]
        for k in range(D2):
          col = jnp.full((16,), k, jnp.int32)
          sv = plsc.load_gather(bs, [r16, col])
          dvv = plsc.load_gather(bd, [r16, col])
          parts[k % 4] = parts[k % 4] + sv * dvv
        s = (parts[0] + parts[1]) + (parts[2] + parts[3])
        obuf[jj, pl.ds(g * 16, 16)] = 1.0 / (1.0 + jnp.exp(-s))
      nj = jj + 4

      @pl.when(nj < CPW)
      def _():
        pltpu.async_copy(ztab.at[srcv.at[nj]], bs, ss)
        pltpu.async_copy(ztab.at[dstv.at[nj]], bd, sd)

  pltpu.sync_copy(obuf, a_out.at[w])


_dec_kernel = pl.kernel(
    _dec_body,
    out_type=(jax.ShapeDtypeStruct((NW, CPW, CHUNK), jnp.float32),
              jax.ShapeDtypeStruct((NPAD, D2), jnp.float32)),
    mesh=_mesh,
    compiler_params=_sc_params,
    scratch_types=[
        pltpu.VMEM_SHARED((NPAD, D2), jnp.float32),
        pltpu.VMEM((CPW, CHUNK), jnp.int32),
        pltpu.VMEM((CPW, CHUNK), jnp.int32),
        pltpu.VMEM((RPT, D2), jnp.float32),
        pltpu.VMEM((RPT, D2), jnp.float32),
        pltpu.VMEM((RPT, D2), jnp.float32),
        pltpu.VMEM((RPT, D2), jnp.float32),
        pltpu.VMEM((1, D2), jnp.float32),
        pltpu.VMEM((CPW, CHUNK), jnp.float32),
    ] + [pltpu.VMEM((CHUNK, D2), jnp.float32) for _ in range(8)]
      + [pltpu.SemaphoreType.DMA for _ in range(8)],
)


BLK = 10240


def _dis_from(dacc):
  deg = dacc[0, :, 0:1] + dacc[1, :, 0:1] + 1.0
  return lax.rsqrt(deg)


def _tc_a_body(x_ref, w1_ref, dacc_ref, o_ref):
  dis = _dis_from(dacc_ref[...])
  h = jnp.dot(x_ref[...], w1_ref[...], preferred_element_type=jnp.float32)
  o_ref[...] = h * dis


def _tc_b_body(a_ref, h1s_ref, dacc_ref, w2_ref, b1_ref, o_ref, disw_ref):
  dis = _dis_from(dacc_ref[...])
  a = a_ref[...]
  z1 = jnp.maximum(dis * (a[0] + a[1] + h1s_ref[...]) + b1_ref[...], 0.0)
  o_ref[...] = jnp.dot(z1, w2_ref[...], preferred_element_type=jnp.float32) * dis
  disw_ref[...] = jnp.broadcast_to(dis, (BLK, D2))


_GRID = (NPAD // BLK,)

_tc_a = pl.pallas_call(
    _tc_a_body,
    grid=_GRID,
    in_specs=[
        pl.BlockSpec((BLK, DIN), lambda i: (i, 0)),
        pl.BlockSpec((DIN, D1), lambda i: (0, 0)),
        pl.BlockSpec((NC, BLK, DEGW), lambda i: (0, i, 0)),
    ],
    out_specs=pl.BlockSpec((BLK, D1), lambda i: (i, 0)),
    out_shape=jax.ShapeDtypeStruct((NPAD, D1), jnp.float32),
)

_tc_b = pl.pallas_call(
    _tc_b_body,
    grid=_GRID,
    in_specs=[
        pl.BlockSpec((NC, BLK, D1), lambda i: (0, i, 0)),
        pl.BlockSpec((BLK, D1), lambda i: (i, 0)),
        pl.BlockSpec((NC, BLK, DEGW), lambda i: (0, i, 0)),
        pl.BlockSpec((D1, D2), lambda i: (0, 0)),
        pl.BlockSpec((1, D1), lambda i: (0, 0)),
    ],
    out_specs=[pl.BlockSpec((BLK, D2), lambda i: (i, 0)),
               pl.BlockSpec((BLK, D2), lambda i: (i, 0))],
    out_shape=[jax.ShapeDtypeStruct((NPAD, D2), jnp.float32),
               jax.ShapeDtypeStruct((NPAD, D2), jnp.float32)],
)


def kernel(x, edge_index, W1, b1, W2, b2):
  f32 = jnp.float32
  x_pad = jnp.zeros((NPAD, DIN), f32).at[:N].set(x)
  rpw = E // NW
  ppw = EPW - rpw
  src_pad = jnp.full((NW, ppw), N, jnp.int32)
  dst_pad = jnp.broadcast_to(N + jnp.arange(ppw, dtype=jnp.int32) % (NPAD - N),
                             (NW, ppw))
  idx = jnp.stack([
      jnp.concatenate([edge_index[0].reshape(NW, rpw), src_pad], axis=1),
      jnp.concatenate([edge_index[1].reshape(NW, rpw), dst_pad], axis=1),
  ]).reshape(2, NW, CPW, CHUNK)

  ones_rows = jnp.ones((CHUNK, DEGW), f32)
  z_deg = jnp.zeros((RPT, DEGW), f32)
  z_64 = jnp.zeros((RPT, D1), f32)
  z_16 = jnp.zeros((RPT, D2), f32)

  dacc = _deg_kernel(idx, ones_rows, z_deg)
  h1s = _tc_a(x_pad, W1, dacc)
  acc1 = _segsum64(idx, h1s, z_64)
  h2s, disw = _tc_b(acc1, h1s, dacc, W2, b1.reshape(1, D1))
  acc2 = _segsum16(idx, h2s, z_16)
  a_out, z2 = _dec_kernel(idx, acc2, h2s, disw, b2.reshape(1, D2))
  a = a_out.reshape(NW, EPW)[:, :E // NW].reshape(-1)
  return z2[:N], a


# speedup vs baseline: 1.0089x; 1.0089x over previous
import jax
import jax.numpy as jnp
from jax import lax
from jax.experimental import pallas as pl
from jax.experimental.pallas import tpu as pltpu
from jax.experimental.pallas import tpu_sc as plsc

N = 10000
E = 320000
DIN = 128
D1 = 64
D2 = 16

NC = 2
NS = 16
NW = NC * NS
CHUNK = 128
CPW = 80
EPW = CPW * CHUNK
EPAD = NW * EPW
NPAD = 10240
RPT = NPAD // NS
DEGW = 8

_mesh = plsc.VectorSubcoreMesh(
    core_axis_name="c", subcore_axis_name="s", num_cores=NC, num_subcores=NS
)
_sc_params = pltpu.CompilerParams(use_tc_tiling_on_sc=False,
                                  needs_layout_passes=False,
                                  disable_bounds_checks=True)


_DEG_GRP = 8


def _deg_body(idx, ones_hbm, zrows, out, acc, dstv, ones_v, sem):
  cid = lax.axis_index("c")
  sid = lax.axis_index("s")
  w = cid * NS + sid
  pltpu.sync_copy(idx.at[1, w], dstv)
  pltpu.sync_copy(ones_hbm, ones_v)
  pltpu.sync_copy(zrows, acc.at[pl.ds(sid * RPT, RPT)])
  plsc.subcore_barrier()

  @pl.loop(0, CPW, step=_DEG_GRP)
  def _(j):
    for b in range(_DEG_GRP):
      pltpu.async_copy(ones_v, acc.at[dstv.at[j + b]], sem, add=True)
    for b in range(_DEG_GRP):
      pltpu.make_async_copy(ones_v, acc.at[dstv.at[j + b]], sem).wait()

  plsc.subcore_barrier()
  pltpu.sync_copy(acc.at[pl.ds(sid * RPT, RPT)],
                  out.at[cid, pl.ds(sid * RPT, RPT)])


_deg_kernel = pl.kernel(
    _deg_body,
    out_type=jax.ShapeDtypeStruct((NC, NPAD, DEGW), jnp.float32),
    mesh=_mesh,
    compiler_params=_sc_params,
    scratch_types=[
        pltpu.VMEM_SHARED((NPAD, DEGW), jnp.float32),
        pltpu.VMEM((CPW, CHUNK), jnp.int32),
        pltpu.VMEM((CHUNK, DEGW), jnp.float32),
        pltpu.SemaphoreType.DMA,
    ],
)


def _make_segsum(D, NBUF, NPH):
  CW = CPW // NPH

  def body(idx, table, zrows, out, acc, tab, srcv, dstv, *bufsem):
    bufs = bufsem[:NBUF]
    gsems = bufsem[NBUF:2 * NBUF]
    ssems = bufsem[2 * NBUF:]
    cid = lax.axis_index("c")
    sid = lax.axis_index("s")
    w = cid * NS + sid
    rows = pl.ds(sid * RPT, RPT)
    pltpu.sync_copy(zrows, acc.at[rows])
    pltpu.sync_copy(table.at[rows], tab.at[rows])
    plsc.subcore_barrier()

    for ph in range(NPH):
      pltpu.sync_copy(idx.at[0, w, pl.ds(ph * CW, CW)], srcv)
      pltpu.sync_copy(idx.at[1, w, pl.ds(ph * CW, CW)], dstv)
      for b in range(NBUF):
        pltpu.async_copy(tab.at[srcv.at[b]], bufs[b], gsems[b])

      @pl.loop(0, CW, step=NBUF)
      def _(j):
        for b in range(NBUF):
          pltpu.make_async_copy(tab.at[srcv.at[j + b]], bufs[b],
                                gsems[b]).wait()
          pltpu.async_copy(bufs[b], acc.at[dstv.at[j + b]], ssems[b], add=True)
        for b in range(NBUF):
          pltpu.make_async_copy(bufs[b], acc.at[dstv.at[j + b]],
                                ssems[b]).wait()
          nj = j + b + NBUF

          @pl.when(nj < CW)
          def _():
            pltpu.async_copy(tab.at[srcv.at[nj]], bufs[b], gsems[b])

    plsc.subcore_barrier()
    pltpu.sync_copy(acc.at[rows], out.at[cid, rows])

  return pl.kernel(
      body,
      out_type=jax.ShapeDtypeStruct((NC, NPAD, D), jnp.float32),
      mesh=_mesh,
      compiler_params=_sc_params,
      scratch_types=[
          pltpu.VMEM_SHARED((NPAD, D), jnp.float32),
          pltpu.VMEM_SHARED((NPAD, D), jnp.float32),
          pltpu.VMEM((CPW // NPH, CHUNK), jnp.int32),
          pltpu.VMEM((CPW // NPH, CHUNK), jnp.int32),
      ] + [pltpu.VMEM((CHUNK, D), jnp.float32) for _ in range(NBUF)]
        + [pltpu.SemaphoreType.DMA for _ in range(2 * NBUF)],
  )


_segsum64 = _make_segsum(D1, 2, 1)
_segsum16 = _make_segsum(D2, 4, 1)


def _dec_body(idx, acc2, h2s, disw, b2w, a_out, z2_out,
              ztab, srcv, dstv, a0v, a1v, h2v, dv, b2v, obuf, *bufsem):
  bufs = bufsem[:8]
  sems = bufsem[8:]
  cid = lax.axis_index("c")
  sid = lax.axis_index("s")
  w = cid * NS + sid
  rows = pl.ds(sid * RPT, RPT)
  pltpu.sync_copy(idx.at[0, w], srcv)
  pltpu.sync_copy(idx.at[1, w], dstv)
  pltpu.sync_copy(acc2.at[0, rows], a0v)
  pltpu.sync_copy(acc2.at[1, rows], a1v)
  pltpu.sync_copy(h2s.at[rows], h2v)
  pltpu.sync_copy(disw.at[rows], dv)
  pltpu.sync_copy(b2w, b2v)
  b2vec = b2v[0, :]

  @pl.loop(0, RPT)
  def _(r):
    z = dv[r, :] * (a0v[r, :] + a1v[r, :] + h2v[r, :]) + b2vec
    a0v[r, :] = jnp.maximum(z, 0.0)

  pltpu.sync_copy(a0v, ztab.at[rows])
  pltpu.sync_copy(a0v, z2_out.at[rows])
  plsc.subcore_barrier()

  rings = tuple((bufs[2 * p], bufs[2 * p + 1], sems[2 * p], sems[2 * p + 1])
                for p in range(4))
  for p in range(4):
    bs, bd, ss, sd = rings[p]
    pltpu.async_copy(ztab.at[srcv.at[p]], bs, ss)
    pltpu.async_copy(ztab.at[dstv.at[p]], bd, sd)

  @pl.loop(0, CPW, step=4)
  def _(j):
    for p in range(4):
      bs, bd, ss, sd = rings[p]
      jj = j + p
      pltpu.make_async_copy(ztab.at[srcv.at[jj]], bs, ss).wait()
      pltpu.make_async_copy(ztab.at[dstv.at[jj]], bd, sd).wait()
      for g in range(CHUNK // 16):
        r16 = lax.iota(jnp.int32, 16) + g * 16
        parts = [jnp.zeros((16,), jnp.float32) for _ in range(4)]
        for k in range(D2):
          col = jnp.full((16,), k, jnp.int32)
          sv = plsc.load_gather(bs, [r16, col])
          dvv = plsc.load_gather(bd, [r16, col])
          parts[k % 4] = parts[k % 4] + sv * dvv
        s = (parts[0] + parts[1]) + (parts[2] + parts[3])
        obuf[jj, pl.ds(g * 16, 16)] = 1.0 / (1.0 + jnp.exp(-s))
      nj = jj + 4

      @pl.when(nj < CPW)
      def _():
        pltpu.async_copy(ztab.at[srcv.at[nj]], bs, ss)
        pltpu.async_copy(ztab.at[dstv.at[nj]], bd, sd)

  pltpu.sync_copy(obuf, a_out.at[w])


_dec_kernel = pl.kernel(
    _dec_body,
    out_type=(jax.ShapeDtypeStruct((NW, CPW, CHUNK), jnp.float32),
              jax.ShapeDtypeStruct((NPAD, D2), jnp.float32)),
    mesh=_mesh,
    compiler_params=_sc_params,
    scratch_types=[
        pltpu.VMEM_SHARED((NPAD, D2), jnp.float32),
        pltpu.VMEM((CPW, CHUNK), jnp.int32),
        pltpu.VMEM((CPW, CHUNK), jnp.int32),
        pltpu.VMEM((RPT, D2), jnp.float32),
        pltpu.VMEM((RPT, D2), jnp.float32),
        pltpu.VMEM((RPT, D2), jnp.float32),
        pltpu.VMEM((RPT, D2), jnp.float32),
        pltpu.VMEM((1, D2), jnp.float32),
        pltpu.VMEM((CPW, CHUNK), jnp.float32),
    ] + [pltpu.VMEM((CHUNK, D2), jnp.float32) for _ in range(8)]
      + [pltpu.SemaphoreType.DMA for _ in range(8)],
)


BLK = 5120


def _dis_from(dacc):
  deg = dacc[0, :, 0:1] + dacc[1, :, 0:1] + 1.0
  return lax.rsqrt(deg)


def _tc_a_body(x_ref, w1_ref, dacc_ref, o_ref):
  dis = _dis_from(dacc_ref[...])
  h = jnp.dot(x_ref[...], w1_ref[...], preferred_element_type=jnp.float32)
  o_ref[...] = h * dis


def _tc_b_body(a_ref, h1s_ref, dacc_ref, w2_ref, b1_ref, o_ref, disw_ref):
  dis = _dis_from(dacc_ref[...])
  a = a_ref[...]
  z1 = jnp.maximum(dis * (a[0] + a[1] + h1s_ref[...]) + b1_ref[...], 0.0)
  o_ref[...] = jnp.dot(z1, w2_ref[...], preferred_element_type=jnp.float32) * dis
  disw_ref[...] = jnp.broadcast_to(dis, (BLK, D2))


_GRID = (NPAD // BLK,)

_tc_a = pl.pallas_call(
    _tc_a_body,
    grid=_GRID,
    in_specs=[
        pl.BlockSpec((BLK, DIN), lambda i: (i, 0)),
        pl.BlockSpec((DIN, D1), lambda i: (0, 0)),
        pl.BlockSpec((NC, BLK, DEGW), lambda i: (0, i, 0)),
    ],
    out_specs=pl.BlockSpec((BLK, D1), lambda i: (i, 0)),
    out_shape=jax.ShapeDtypeStruct((NPAD, D1), jnp.float32),
)

_tc_b = pl.pallas_call(
    _tc_b_body,
    grid=_GRID,
    in_specs=[
        pl.BlockSpec((NC, BLK, D1), lambda i: (0, i, 0)),
        pl.BlockSpec((BLK, D1), lambda i: (i, 0)),
        pl.BlockSpec((NC, BLK, DEGW), lambda i: (0, i, 0)),
        pl.BlockSpec((D1, D2), lambda i: (0, 0)),
        pl.BlockSpec((1, D1), lambda i: (0, 0)),
    ],
    out_specs=[pl.BlockSpec((BLK, D2), lambda i: (i, 0)),
               pl.BlockSpec((BLK, D2), lambda i: (i, 0))],
    out_shape=[jax.ShapeDtypeStruct((NPAD, D2), jnp.float32),
               jax.ShapeDtypeStruct((NPAD, D2), jnp.float32)],
)


def kernel(x, edge_index, W1, b1, W2, b2):
  f32 = jnp.float32
  x_pad = jnp.zeros((NPAD, DIN), f32).at[:N].set(x)
  rpw = E // NW
  ppw = EPW - rpw
  src_pad = jnp.full((NW, ppw), N, jnp.int32)
  dst_pad = jnp.broadcast_to(N + jnp.arange(ppw, dtype=jnp.int32) % (NPAD - N),
                             (NW, ppw))
  idx = jnp.stack([
      jnp.concatenate([edge_index[0].reshape(NW, rpw), src_pad], axis=1),
      jnp.concatenate([edge_index[1].reshape(NW, rpw), dst_pad], axis=1),
  ]).reshape(2, NW, CPW, CHUNK)

  ones_rows = jnp.ones((CHUNK, DEGW), f32)
  z_deg = jnp.zeros((RPT, DEGW), f32)
  z_64 = jnp.zeros((RPT, D1), f32)
  z_16 = jnp.zeros((RPT, D2), f32)

  dacc = _deg_kernel(idx, ones_rows, z_deg)
  h1s = _tc_a(x_pad, W1, dacc)
  acc1 = _segsum64(idx, h1s, z_64)
  h2s, disw = _tc_b(acc1, h1s, dacc, W2, b1.reshape(1, D1))
  acc2 = _segsum16(idx, h2s, z_16)
  a_out, z2 = _dec_kernel(idx, acc2, h2s, disw, b2.reshape(1, D2))
  a = a_out.reshape(NW, EPW)[:, :E // NW].reshape(-1)
  return z2[:N], a
